# baseline (speedup 1.0000x reference)
import functools
import math

import jax
import jax.numpy as jnp
from jax import lax
from jax.experimental import pallas as pl
from jax.experimental.pallas import tpu as pltpu
from jax.experimental.pallas import tpu_sc as plsc

B = 1024
V = 100000
M = 0.5
COS_M = math.cos(M)
SIN_M = math.sin(M)


_NC = 2
_NS = 16
_NW = _NC * _NS
_BPW = B // _NW


@functools.cache
def _build_sc_gather():
    mesh = plsc.VectorSubcoreMesh(core_axis_name="c", subcore_axis_name="s")

    @functools.partial(
        pl.kernel,
        mesh=mesh,
        out_type=jax.ShapeDtypeStruct((B,), jnp.float32),
        scratch_types=[
            pltpu.VMEM((_BPW,), jnp.int32),
            pltpu.VMEM((_BPW,), jnp.int32),
            pltpu.VMEM((_BPW,), jnp.float32),
            pltpu.SemaphoreType.DMA,
        ],
    )
    def gather_kernel(flat_hbm, tgt_hbm, out_hbm, tgt_v, idx_v, c_v, sem):
        wid = lax.axis_index("s") * _NC + lax.axis_index("c")
        base = wid * _BPW
        pltpu.sync_copy(tgt_hbm.at[pl.ds(base, _BPW)], tgt_v)
        for j in range(_BPW // 16):
            t = tgt_v[pl.ds(j * 16, 16)]
            rows = lax.iota(jnp.int32, 16) + (base + j * 16)
            idx_v[pl.ds(j * 16, 16)] = rows * V + t
        pltpu.async_copy(flat_hbm.at[idx_v], c_v, sem).wait()
        pltpu.sync_copy(c_v, out_hbm.at[pl.ds(base, _BPW)])

    return gather_kernel


_RB = 8
_NR = B // _RB
_CH = 2048
_NFULL = V // _CH
_TAIL0 = _NFULL * _CH
_TAIL_128 = ((V - _TAIL0) // 128) * 128

_NBUF = 2
_NSPLIT = 4
_SEG = [(0, 25088), (25088, 25088), (50176, 25088), (75264, V - 75264)]


def _start_copies(x_hbm, buf, sems, step, slot):
    for k, (off, ln) in enumerate(_SEG):
        pltpu.make_async_copy(
            x_hbm.at[pl.ds(step * _RB, _RB), pl.ds(off, ln)],
            buf.at[slot, :, pl.ds(off, ln)],
            sems.at[slot, k],
        ).start()


def _wait_copies(x_hbm, buf, sems, step, slot):
    for k, (off, ln) in enumerate(_SEG):
        pltpu.make_async_copy(
            x_hbm.at[pl.ds(step * _RB, _RB), pl.ds(off, ln)],
            buf.at[slot, :, pl.ds(off, ln)],
            sems.at[slot, k],
        ).wait()


def _tc_body(x_hbm, c_ref, out_ref, buf, sems):
    i = pl.program_id(0)
    slot = lax.rem(i, _NBUF)

    @pl.when(i == 0)
    def _prime():
        _start_copies(x_hbm, buf, sems, 0, 0)

    @pl.when(i + 1 < _NR)
    def _prefetch():
        _start_copies(x_hbm, buf, sems, i + 1, lax.rem(i + 1, _NBUF))

    _wait_copies(x_hbm, buf, sems, i, slot)

    x = buf[slot]
    acc = jnp.exp(x[:, 0:_CH])
    for k in range(1, _NFULL):
        acc += jnp.exp(x[:, k * _CH:(k + 1) * _CH])
    rowsum = jnp.sum(acc, axis=1, keepdims=True)
    rowsum += jnp.sum(jnp.exp(x[:, _TAIL0:_TAIL0 + _TAIL_128]),
                      axis=1, keepdims=True)
    rowsum += jnp.sum(jnp.exp(x[:, _TAIL0 + _TAIL_128:V]),
                      axis=1, keepdims=True)

    c = c_ref[...]
    sin_t = jnp.sqrt(jnp.maximum(1.0 - c * c, 0.0))
    new_cos = c * COS_M - sin_t * SIN_M
    stot = rowsum - jnp.exp(c) + jnp.exp(new_cos)
    nll = jnp.log(stot) - new_cos
    partial = jnp.sum(nll) / B

    @pl.when(i == 0)
    def _init():
        out_ref[0, 0] = partial

    @pl.when(i > 0)
    def _accum():
        out_ref[0, 0] += partial


def _tc_loss(inp, c):
    return pl.pallas_call(
        _tc_body,
        grid=(_NR,),
        in_specs=[
            pl.BlockSpec(memory_space=pl.ANY),
            pl.BlockSpec((_RB, 1), lambda i: (i, 0)),
        ],
        out_specs=pl.BlockSpec(memory_space=pltpu.SMEM),
        out_shape=jax.ShapeDtypeStruct((1, 1), jnp.float32),
        scratch_shapes=[
            pltpu.VMEM((_NBUF, _RB, V), jnp.float32),
            pltpu.SemaphoreType.DMA((_NBUF, _NSPLIT)),
        ],
    )(inp, c)


def kernel(input, target):
    flat = input.reshape(B * V)
    c = _build_sc_gather()(flat, target.astype(jnp.int32))
    out = _tc_loss(input, c.reshape(B, 1))
    return out[0, 0]


# speedup vs baseline: 1.6498x; 1.6498x over previous
import functools
import math

import jax
import jax.numpy as jnp
from jax import lax
from jax.experimental import pallas as pl
from jax.experimental.pallas import tpu as pltpu
from jax.experimental.pallas import tpu_sc as plsc

B = 1024
V = 100000
M = 0.5
COS_M = math.cos(M)
SIN_M = math.sin(M)

_NC = 2
_NS = 16
_NW = _NC * _NS
_RPT = B // _NW
_G = 8
_NGRP = _RPT // _G
_CW = 6144
_NCH = 16
_SCCOLS = _CW * _NCH
_TAILW = V - _SCCOLS
_UNR = 16
_ROWIT = _CW // (16 * _UNR)


def _row_sums(buf, accs, cvecs, rels):
    lane = lax.iota(jnp.int32, 16)
    outa, outc = [], []
    for r in range(_G):
        def body(i, ac, r=r):
            a, c = ac
            base = i * (16 * _UNR)
            for u in range(_UNR):
                o = base + u * 16
                v = buf[r, pl.ds(o, 16)]
                a = a + jnp.exp(v)
                c = jnp.where(lane == rels[r] - o, v, c)
            return (a, c)
        a, c = lax.fori_loop(0, _ROWIT, body, (accs[r], cvecs[r]))
        outa.append(a)
        outc.append(c)
    return tuple(outa), tuple(outc)


@functools.cache
def _build_sc_sumexp():
    mesh = plsc.VectorSubcoreMesh(core_axis_name="c", subcore_axis_name="s")

    @functools.partial(
        pl.kernel,
        mesh=mesh,
        out_type=(
            jax.ShapeDtypeStruct((B, 16), jnp.float32),
            jax.ShapeDtypeStruct((B, 16), jnp.float32),
        ),
        scratch_types=[
            pltpu.VMEM((_RPT, 16), jnp.int32),
            pltpu.VMEM((_RPT, 16), jnp.float32),
            pltpu.VMEM((_RPT, 16), jnp.float32),
            pltpu.VMEM((_G, _CW), jnp.float32),
            pltpu.VMEM((_G, _CW), jnp.float32),
            pltpu.SemaphoreType.DMA,
            pltpu.SemaphoreType.DMA,
        ],
    )
    def sc_kernel(x_hbm, tgtb_hbm, s16_out, c16_out,
                  tgtb_v, c16, srow, buf_a, buf_b, sem_a, sem_b):
        wid = lax.axis_index("s") * _NC + lax.axis_index("c")
        base = wid * _RPT
        pltpu.sync_copy(tgtb_hbm.at[pl.ds(base, _RPT)], tgtb_v)
        zero16 = jnp.zeros((16,), jnp.float32)

        def start(rows0, ch, buf, sem):
            pltpu.make_async_copy(
                x_hbm.at[pl.ds(rows0, _G), pl.ds(ch * _CW, _CW)],
                buf, sem).start()

        def wait(rows0, ch, buf, sem):
            pltpu.make_async_copy(
                x_hbm.at[pl.ds(rows0, _G), pl.ds(ch * _CW, _CW)],
                buf, sem).wait()

        for g in range(_NGRP):
            rows0 = base + g * _G
            tvecs = [tgtb_v[g * _G + r] for r in range(_G)]
            start(rows0, 0, buf_a, sem_a)

            def pair_body(p, carry, rows0=rows0, tvecs=tvecs):
                accs, cvecs = carry
                off_a = 2 * p * _CW
                off_b = (2 * p + 1) * _CW
                start(rows0, 2 * p + 1, buf_b, sem_b)
                wait(rows0, 2 * p, buf_a, sem_a)
                rels_a = [tvecs[r] - off_a for r in range(_G)]
                accs, cvecs = _row_sums(buf_a, accs, cvecs, rels_a)

                @pl.when(p + 1 < _NCH // 2)
                def _next():
                    start(rows0, 2 * p + 2, buf_a, sem_a)

                wait(rows0, 2 * p + 1, buf_b, sem_b)
                rels_b = [tvecs[r] - off_b for r in range(_G)]
                accs, cvecs = _row_sums(buf_b, accs, cvecs, rels_b)
                return (accs, cvecs)

            accs, cvecs = lax.fori_loop(
                0, _NCH // 2, pair_body,
                (tuple(zero16 for _ in range(_G)),
                 tuple(zero16 for _ in range(_G))))

            for r in range(_G):
                srow[g * _G + r] = accs[r]
                c16[g * _G + r] = cvecs[r]

        pltpu.sync_copy(srow, s16_out.at[pl.ds(base, _RPT)])
        pltpu.sync_copy(c16, c16_out.at[pl.ds(base, _RPT)])

    return sc_kernel


_TB = 2048


def _combine_body(s16_ref, c16_ref, tgt_ref, xtail_ref, out_ref):
    xt = xtail_ref[...]
    colid = lax.broadcasted_iota(jnp.int32, (B, _TB), 1)
    valid = colid < _TAILW
    e = jnp.where(valid, jnp.exp(xt), 0.0)
    tail_sum = jnp.sum(e, axis=1, keepdims=True)

    trel = tgt_ref[...] - _SCCOLS
    hit = (colid == trel) & valid
    c_tail = jnp.sum(jnp.where(hit, xt, 0.0), axis=1, keepdims=True)
    c_sc = jnp.sum(c16_ref[...], axis=1, keepdims=True)
    c = jnp.where(trel >= 0, c_tail, c_sc)

    s0 = jnp.sum(s16_ref[...], axis=1, keepdims=True) + tail_sum
    sin_t = jnp.sqrt(jnp.maximum(1.0 - c * c, 0.0))
    new_cos = c * COS_M - sin_t * SIN_M
    stot = s0 - jnp.exp(c) + jnp.exp(new_cos)
    nll = jnp.log(stot) - new_cos
    out_ref[0, 0] = jnp.sum(nll) / B


def _tc_combine(s16, c16, tgt, inp):
    return pl.pallas_call(
        _combine_body,
        grid=(1,),
        in_specs=[
            pl.BlockSpec((B, 16), lambda i: (0, 0)),
            pl.BlockSpec((B, 16), lambda i: (0, 0)),
            pl.BlockSpec((B, 1), lambda i: (0, 0)),
            pl.BlockSpec((B, _TB), lambda i: (0, _SCCOLS // _TB)),
        ],
        out_specs=pl.BlockSpec(memory_space=pltpu.SMEM),
        out_shape=jax.ShapeDtypeStruct((1, 1), jnp.float32),
    )(s16, c16, tgt, inp)


def kernel(input, target):
    tgt = target.astype(jnp.int32)
    tgt_b = jnp.broadcast_to(tgt.reshape(B, 1), (B, 16))
    s16, c16 = _build_sc_sumexp()(input, tgt_b)
    out = _tc_combine(s16, c16, tgt.reshape(B, 1), input)
    return out[0, 0]


# speedup vs baseline: 2.1022x; 1.2742x over previous
import functools
import math

import jax
import jax.numpy as jnp
from jax import lax
from jax.experimental import pallas as pl
from jax.experimental.pallas import tpu as pltpu
from jax.experimental.pallas import tpu_sc as plsc

B = 1024
V = 100000
M = 0.5
COS_M = math.cos(M)
SIN_M = math.sin(M)

_TCR = 512
_SCR = B - _TCR


_NC = 2
_NS = 16
_NW = _NC * _NS
_RPT = _SCR // _NW
_G = 8
_NGRP = _RPT // _G
_CW = 6144
_NCH = 16
_SCCOLS = _CW * _NCH
_TAILW = V - _SCCOLS
_UNR = 16
_ROWIT = _CW // (16 * _UNR)


def _row_sums(buf, accs, cvecs, rels):
    lane = lax.iota(jnp.int32, 16)
    outa, outc = [], []
    for r in range(_G):
        def body(i, ac, r=r):
            a, c = ac
            base = i * (16 * _UNR)
            for u in range(_UNR):
                o = base + u * 16
                v = buf[r, pl.ds(o, 16)]
                a = a + jnp.exp(v)
                c = jnp.where(lane == rels[r] - o, v, c)
            return (a, c)
        a, c = lax.fori_loop(0, _ROWIT, body, (accs[r], cvecs[r]))
        outa.append(a)
        outc.append(c)
    return tuple(outa), tuple(outc)


@functools.cache
def _build_sc_sumexp():
    mesh = plsc.VectorSubcoreMesh(core_axis_name="c", subcore_axis_name="s")

    @functools.partial(
        pl.kernel,
        mesh=mesh,
        out_type=(
            jax.ShapeDtypeStruct((_SCR, 16), jnp.float32),
            jax.ShapeDtypeStruct((_SCR, 16), jnp.float32),
        ),
        scratch_types=[
            pltpu.VMEM((_RPT, 16), jnp.int32),
            pltpu.VMEM((_RPT, 16), jnp.float32),
            pltpu.VMEM((_RPT, 16), jnp.float32),
            pltpu.VMEM((_G, _CW), jnp.float32),
            pltpu.VMEM((_G, _CW), jnp.float32),
            pltpu.SemaphoreType.DMA,
            pltpu.SemaphoreType.DMA,
        ],
    )
    def sc_kernel(x_hbm, tgtb_hbm, s16_out, c16_out,
                  tgtb_v, c16, srow, buf_a, buf_b, sem_a, sem_b):
        wid = lax.axis_index("s") * _NC + lax.axis_index("c")
        base = wid * _RPT
        pltpu.sync_copy(tgtb_hbm.at[pl.ds(_TCR + base, _RPT)], tgtb_v)
        zero16 = jnp.zeros((16,), jnp.float32)

        def start(rows0, ch, buf, sem):
            pltpu.make_async_copy(
                x_hbm.at[pl.ds(rows0, _G), pl.ds(ch * _CW, _CW)],
                buf, sem).start()

        def wait(rows0, ch, buf, sem):
            pltpu.make_async_copy(
                x_hbm.at[pl.ds(rows0, _G), pl.ds(ch * _CW, _CW)],
                buf, sem).wait()

        for g in range(_NGRP):
            rows0 = _TCR + base + g * _G
            tvecs = [tgtb_v[g * _G + r] for r in range(_G)]
            start(rows0, 0, buf_a, sem_a)

            def pair_body(p, carry, rows0=rows0, tvecs=tvecs):
                accs, cvecs = carry
                off_a = 2 * p * _CW
                off_b = (2 * p + 1) * _CW
                start(rows0, 2 * p + 1, buf_b, sem_b)
                wait(rows0, 2 * p, buf_a, sem_a)
                rels_a = [tvecs[r] - off_a for r in range(_G)]
                accs, cvecs = _row_sums(buf_a, accs, cvecs, rels_a)

                @pl.when(p + 1 < _NCH // 2)
                def _next():
                    start(rows0, 2 * p + 2, buf_a, sem_a)

                wait(rows0, 2 * p + 1, buf_b, sem_b)
                rels_b = [tvecs[r] - off_b for r in range(_G)]
                accs, cvecs = _row_sums(buf_b, accs, cvecs, rels_b)
                return (accs, cvecs)

            accs, cvecs = lax.fori_loop(
                0, _NCH // 2, pair_body,
                (tuple(zero16 for _ in range(_G)),
                 tuple(zero16 for _ in range(_G))))

            for r in range(_G):
                srow[g * _G + r] = accs[r]
                c16[g * _G + r] = cvecs[r]

        pltpu.sync_copy(srow, s16_out.at[pl.ds(base, _RPT)])
        pltpu.sync_copy(c16, c16_out.at[pl.ds(base, _RPT)])

    return sc_kernel


_RB = 8
_NRB = _TCR // _RB
_CH = 2048
_NFULL = V // _CH
_T0 = _NFULL * _CH
_T128 = ((V - _T0) // 128) * 128
_NBUF = 2
_SEG = [(0, 25088), (25088, 25088), (50176, 25088), (75264, V - 75264)]


def _tc_start(x_hbm, buf, sems, step, slot):
    for k, (off, ln) in enumerate(_SEG):
        pltpu.make_async_copy(
            x_hbm.at[pl.ds(step * _RB, _RB), pl.ds(off, ln)],
            buf.at[slot, :, pl.ds(off, ln)],
            sems.at[slot, k],
        ).start()


def _tc_wait(x_hbm, buf, sems, step, slot):
    for k, (off, ln) in enumerate(_SEG):
        pltpu.make_async_copy(
            x_hbm.at[pl.ds(step * _RB, _RB), pl.ds(off, ln)],
            buf.at[slot, :, pl.ds(off, ln)],
            sems.at[slot, k],
        ).wait()


def _tc_stream_body(x_hbm, tgt_ref, s_ref, c_ref, buf, sems):
    i = pl.program_id(0)
    slot = lax.rem(i, _NBUF)

    @pl.when(i == 0)
    def _prime():
        _tc_start(x_hbm, buf, sems, 0, 0)

    @pl.when(i + 1 < _NRB)
    def _prefetch():
        _tc_start(x_hbm, buf, sems, i + 1, lax.rem(i + 1, _NBUF))

    _tc_wait(x_hbm, buf, sems, i, slot)

    x = buf[slot]
    t8 = tgt_ref[...]
    ch_iota = lax.broadcasted_iota(jnp.int32, (_RB, _CH), 1)

    acc = jnp.exp(x[:, 0:_CH])
    cacc = jnp.where(ch_iota == t8, x[:, 0:_CH], 0.0)
    for k in range(1, _NFULL):
        xs = x[:, k * _CH:(k + 1) * _CH]
        acc += jnp.exp(xs)
        cacc += jnp.where(ch_iota + k * _CH == t8, xs, 0.0)
    rowsum = jnp.sum(acc, axis=1, keepdims=True)
    crow = jnp.sum(cacc, axis=1, keepdims=True)

    xs = x[:, _T0:_T0 + _T128]
    tl_iota = lax.broadcasted_iota(jnp.int32, (_RB, _T128), 1) + _T0
    rowsum += jnp.sum(jnp.exp(xs), axis=1, keepdims=True)
    crow += jnp.sum(jnp.where(tl_iota == t8, xs, 0.0), axis=1, keepdims=True)

    xs = x[:, _T0 + _T128:V]
    rm_iota = (lax.broadcasted_iota(jnp.int32, (_RB, V - _T0 - _T128), 1)
               + _T0 + _T128)
    rowsum += jnp.sum(jnp.exp(xs), axis=1, keepdims=True)
    crow += jnp.sum(jnp.where(rm_iota == t8, xs, 0.0), axis=1, keepdims=True)

    s_ref[...] = rowsum
    c_ref[...] = crow


def _tc_stream(inp, tgt):
    return pl.pallas_call(
        _tc_stream_body,
        grid=(_NRB,),
        in_specs=[
            pl.BlockSpec(memory_space=pl.ANY),
            pl.BlockSpec((_RB, 1), lambda i: (i, 0)),
        ],
        out_specs=[
            pl.BlockSpec((_RB, 1), lambda i: (i, 0)),
            pl.BlockSpec((_RB, 1), lambda i: (i, 0)),
        ],
        out_shape=[
            jax.ShapeDtypeStruct((_TCR, 1), jnp.float32),
            jax.ShapeDtypeStruct((_TCR, 1), jnp.float32),
        ],
        scratch_shapes=[
            pltpu.VMEM((_NBUF, _RB, V), jnp.float32),
            pltpu.SemaphoreType.DMA((_NBUF, len(_SEG))),
        ],
    )(inp, tgt)


_TB = 2048


def _combine_body(s16_ref, c16_ref, stc_ref, ctc_ref, tgt_ref, xtail_ref,
                  out_ref):
    xt = xtail_ref[...]
    colid = lax.broadcasted_iota(jnp.int32, (_SCR, _TB), 1)
    valid = colid < _TAILW
    e = jnp.where(valid, jnp.exp(xt), 0.0)
    tail_sum = jnp.sum(e, axis=1, keepdims=True)

    trel = tgt_ref[...] - _SCCOLS
    hit = (colid == trel) & valid
    c_tail = jnp.sum(jnp.where(hit, xt, 0.0), axis=1, keepdims=True)
    c_sc = jnp.sum(c16_ref[...], axis=1, keepdims=True)
    c_sc = jnp.where(trel >= 0, c_tail, c_sc)
    s_sc = jnp.sum(s16_ref[...], axis=1, keepdims=True) + tail_sum

    def nll_sum(s, c):
        sin_t = jnp.sqrt(jnp.maximum(1.0 - c * c, 0.0))
        new_cos = c * COS_M - sin_t * SIN_M
        stot = s - jnp.exp(c) + jnp.exp(new_cos)
        return jnp.sum(jnp.log(stot) - new_cos)

    out_ref[0, 0] = (nll_sum(s_sc, c_sc)
                     + nll_sum(stc_ref[...], ctc_ref[...])) / B


def _tc_combine(s16, c16, s_tc, c_tc, tgt_sc, inp):
    return pl.pallas_call(
        _combine_body,
        grid=(1,),
        in_specs=[
            pl.BlockSpec((_SCR, 16), lambda i: (0, 0)),
            pl.BlockSpec((_SCR, 16), lambda i: (0, 0)),
            pl.BlockSpec((_TCR, 1), lambda i: (0, 0)),
            pl.BlockSpec((_TCR, 1), lambda i: (0, 0)),
            pl.BlockSpec((_SCR, 1), lambda i: (0, 0)),
            pl.BlockSpec((_SCR, _TB), lambda i: (_TCR // _SCR, _SCCOLS // _TB)),
        ],
        out_specs=pl.BlockSpec(memory_space=pltpu.SMEM),
        out_shape=jax.ShapeDtypeStruct((1, 1), jnp.float32),
    )(s16, c16, s_tc, c_tc, tgt_sc, inp)


def kernel(input, target):
    tgt = target.astype(jnp.int32).reshape(B, 1)
    tgt_b = jnp.broadcast_to(tgt, (B, 16))
    s16, c16 = _build_sc_sumexp()(input, tgt_b)
    s_tc, c_tc = _tc_stream(input, tgt[:_TCR])
    out = _tc_combine(s16, c16, s_tc, c_tc, tgt[_TCR:], input)
    return out[0, 0]
